# baseline (speedup 1.0000x reference)
import dataclasses
import functools

import jax
import jax.numpy as jnp
from jax import lax
from jax.experimental import pallas as pl
from jax.experimental.pallas import tpu as pltpu
from jax.experimental.pallas import tpu_sc as plsc

_NC = 2
_NS = 16
_L = 16
_NW = _NC * _NS

_CHUNKS = (1024, 2048, 2048, 3072)


def _wid():
    return lax.axis_index("s") * _NC + lax.axis_index("c")


def _sc_compiler_params():
    cp = pltpu.CompilerParams()
    if "needs_layout_passes" in pltpu.CompilerParams.__dataclass_fields__:
        cp = dataclasses.replace(cp, needs_layout_passes=False)
    return cp


def _make_sort_kernel(M, E):
    chunk = M // _NW
    n_vec_total = M // _L
    n_vec_chunk = chunk // _L
    mesh = plsc.VectorSubcoreMesh(core_axis_name="c", subcore_axis_name="s")

    @functools.partial(
        pl.kernel,
        out_type=jax.ShapeDtypeStruct((M,), jnp.int32),
        mesh=mesh,
        scratch_types=[
            pltpu.VMEM((M,), jnp.int32),
            pltpu.VMEM((chunk,), jnp.int32),
            pltpu.VMEM((2 * _L,), jnp.int32),
        ],
        compiler_params=_sc_compiler_params(),
    )
    def sort_kernel(e_hbm, pos_hbm, e_v, pos_v, acc_v):
        w = _wid()
        lanes = lax.iota(jnp.int32, _L)
        onehots = [
            jnp.where(lanes == v, jnp.int32(1), jnp.int32(0)) for v in range(E)
        ]
        pltpu.sync_copy(e_hbm, e_v)

        first_own = w * n_vec_chunk
        acc_v[pl.ds(0, _L)] = jnp.zeros((_L,), jnp.int32)
        acc_v[pl.ds(_L, _L)] = jnp.zeros((_L,), jnp.int32)

        @pl.loop(0, n_vec_total)
        def _(t):
            ev = e_v[pl.ds(t * _L, _L)]
            is_before = jnp.where(t < first_own, jnp.int32(1), jnp.int32(0))
            tot = acc_v[pl.ds(0, _L)]
            bef = acc_v[pl.ds(_L, _L)]
            for v in range(E):
                cnt = plsc.all_reduce_population_count(ev == v)
                tot = tot + cnt * onehots[v]
                bef = bef + (cnt * is_before) * onehots[v]
            acc_v[pl.ds(0, _L)] = tot
            acc_v[pl.ds(_L, _L)] = bef

        total = acc_v[pl.ds(0, _L)]
        before = acc_v[pl.ds(_L, _L)]
        start0 = (plsc.cumsum(total) - total) + before

        def body(t2, start):
            ev = e_v[pl.ds((first_own + t2) * _L, _L)]
            pos_vec = jnp.zeros((_L,), jnp.int32)
            for v in range(E):
                m = ev == v
                mi = jnp.where(m, jnp.int32(1), jnp.int32(0))
                incl = plsc.cumsum(mi)
                base_v = jnp.sum(start * onehots[v])
                pos_vec = jnp.where(m, base_v + incl - 1, pos_vec)
                cnt = plsc.all_reduce_population_count(m)
                start = start + cnt * onehots[v]
            pos_v[pl.ds(t2 * _L, _L)] = pos_vec
            return start

        lax.fori_loop(0, n_vec_chunk, body, start0)

        pltpu.sync_copy(pos_v, pos_hbm.at[pl.ds(w * chunk, chunk)])

    return sort_kernel


def _make_gather_kernel(M, H, start, Mc):
    rows_w = Mc // _NW
    cb = 16
    n_cb = rows_w // cb
    n_vec_total = M // _L
    mesh = plsc.VectorSubcoreMesh(core_axis_name="c", subcore_axis_name="s")

    @functools.partial(
        pl.kernel,
        out_type=jax.ShapeDtypeStruct((Mc, H), jnp.float32),
        mesh=mesh,
        scratch_types=(
            [pltpu.VMEM((M,), jnp.int32),
             pltpu.VMEM((rows_w,), jnp.int32)]
            + [pltpu.VMEM((cb, H), jnp.float32)] * 2
            + [pltpu.VMEM((cb,), jnp.int32)] * 2
            + [pltpu.SemaphoreType.DMA] * 4
        ),
        compiler_params=_sc_compiler_params(),
    )
    def gather_kernel(x_hbm, pos_hbm, xs_hbm, pos_v, perm_v, *rest):
        bufs = rest[0:2]
        idxs = rest[2:4]
        gsems = rest[4:6]
        ssems = rest[6:8]
        w = _wid()
        lanes = lax.iota(jnp.int32, _L)
        base = start + w * rows_w
        pltpu.sync_copy(pos_hbm, pos_v)

        @pl.loop(0, n_vec_total)
        def _(t):
            pv = pos_v[pl.ds(t * _L, _L)]
            rel = pv - base
            m = (rel >= 0) & (rel < rows_w)
            relc = jnp.where(m, rel, 0)
            plsc.store_scatter(perm_v, [relc], lanes + t * _L, mask=m)

        gathers = [None, None]
        stores = [None] * n_cb
        for c in range(min(2, n_cb)):
            idxs[c][...] = perm_v[pl.ds(c * cb, cb)]
            gathers[c] = pltpu.async_copy(
                x_hbm.at[idxs[c]], bufs[c], gsems[c])
        for c in range(n_cb):
            b = c & 1
            gathers[b].wait()
            stores[c] = pltpu.async_copy(
                bufs[b], xs_hbm.at[pl.ds(w * rows_w + c * cb, cb)], ssems[b])
            nxt = c + 2
            if nxt < n_cb:
                stores[c].wait()
                idxs[b][...] = perm_v[pl.ds(nxt * cb, cb)]
                gathers[b] = pltpu.async_copy(
                    x_hbm.at[idxs[b]], bufs[b], gsems[b])
        for c in range(max(0, n_cb - 2), n_cb):
            if stores[c] is not None:
                stores[c].wait()

    return gather_kernel


def _mm_body(x_ref, w_ref, be_ref, s_ref, o_ref):
    xb = x_ref[...].astype(jnp.bfloat16)
    acc = jnp.dot(xb, w_ref[...], preferred_element_type=jnp.float32)
    o_ref[...] = (acc + be_ref[...]) * s_ref[...]


def _mm_body_acc(x_ref, w_ref, be_ref, s_ref, prev_ref, o_ref):
    del prev_ref
    _mm_body(x_ref, w_ref, be_ref, s_ref, o_ref)


def _expert_matmul_chunk(xs, We_bf, be2, p2, prev, start, bm=1024):
    Mc, H = xs.shape
    M = p2.shape[0]
    blocks = Mc // bm
    b0 = start // bm
    in_specs = [
        pl.BlockSpec((bm, H), lambda i: (i, 0)),
        pl.BlockSpec((H, H), lambda i: (0, 0)),
        pl.BlockSpec((1, H), lambda i: (0, 0)),
        pl.BlockSpec((bm, 1), lambda i, b0=b0: (b0 + i, 0)),
    ]
    args = [xs, We_bf, be2, p2]
    if prev is None:
        body = _mm_body
        aliases = {}
    else:
        body = _mm_body_acc
        in_specs.append(pl.BlockSpec(memory_space=pl.ANY))
        args.append(prev)
        aliases = {4: 0}
    return pl.pallas_call(
        body,
        grid=(blocks,),
        in_specs=in_specs,
        out_specs=pl.BlockSpec((bm, H), lambda i, b0=b0: (b0 + i, 0)),
        out_shape=jax.ShapeDtypeStruct((M, H), jnp.float32),
        input_output_aliases=aliases,
    )(*args)


def kernel(x, Wg, bg, We, be):
    B, S, H = x.shape
    E = Wg.shape[1]
    M = B * S
    hs = x.reshape(M, H)

    router_logits = hs @ Wg + bg
    normalized_logits = jax.nn.softmax(router_logits, axis=1)
    best = jnp.argmax(normalized_logits, axis=1)
    p = jnp.max(normalized_logits, axis=1)

    e = best.astype(jnp.int32)
    pos = _make_sort_kernel(M, E)(e)
    We_bf = We.astype(jnp.bfloat16)
    be2 = be.reshape(1, H)
    p2 = p.reshape(M, 1)

    out = None
    start = 0
    for Mc in _CHUNKS:
        xs_k = _make_gather_kernel(M, H, start, Mc)(hs, pos)
        out = _expert_matmul_chunk(xs_k, We_bf, be2, p2, out, start)
        start += Mc
    return out.reshape(B, S, H)


# speedup vs baseline: 1.0261x; 1.0261x over previous
import dataclasses
import functools

import jax
import jax.numpy as jnp
from jax import lax
from jax.experimental import pallas as pl
from jax.experimental.pallas import tpu as pltpu
from jax.experimental.pallas import tpu_sc as plsc

_NC = 2
_NS = 16
_L = 16
_NW = _NC * _NS

_CHUNKS = (2048, 2048, 2048, 2048)


def _wid():
    return lax.axis_index("s") * _NC + lax.axis_index("c")


def _sc_compiler_params():
    cp = pltpu.CompilerParams()
    if "needs_layout_passes" in pltpu.CompilerParams.__dataclass_fields__:
        cp = dataclasses.replace(cp, needs_layout_passes=False)
    return cp


def _make_sort_kernel(M, E):
    chunk = M // _NW
    n_vec_total = M // _L
    n_vec_chunk = chunk // _L
    mesh = plsc.VectorSubcoreMesh(core_axis_name="c", subcore_axis_name="s")

    @functools.partial(
        pl.kernel,
        out_type=jax.ShapeDtypeStruct((M,), jnp.int32),
        mesh=mesh,
        scratch_types=[
            pltpu.VMEM((M,), jnp.int32),
            pltpu.VMEM((chunk,), jnp.int32),
            pltpu.VMEM((2 * _L,), jnp.int32),
        ],
        compiler_params=_sc_compiler_params(),
    )
    def sort_kernel(e_hbm, pos_hbm, e_v, pos_v, acc_v):
        w = _wid()
        lanes = lax.iota(jnp.int32, _L)
        onehots = [
            jnp.where(lanes == v, jnp.int32(1), jnp.int32(0)) for v in range(E)
        ]
        pltpu.sync_copy(e_hbm, e_v)

        first_own = w * n_vec_chunk
        acc_v[pl.ds(0, _L)] = jnp.zeros((_L,), jnp.int32)
        acc_v[pl.ds(_L, _L)] = jnp.zeros((_L,), jnp.int32)

        @pl.loop(0, n_vec_total)
        def _(t):
            ev = e_v[pl.ds(t * _L, _L)]
            is_before = jnp.where(t < first_own, jnp.int32(1), jnp.int32(0))
            tot = acc_v[pl.ds(0, _L)]
            bef = acc_v[pl.ds(_L, _L)]
            for v in range(E):
                cnt = plsc.all_reduce_population_count(ev == v)
                tot = tot + cnt * onehots[v]
                bef = bef + (cnt * is_before) * onehots[v]
            acc_v[pl.ds(0, _L)] = tot
            acc_v[pl.ds(_L, _L)] = bef

        total = acc_v[pl.ds(0, _L)]
        before = acc_v[pl.ds(_L, _L)]
        start0 = (plsc.cumsum(total) - total) + before

        def body(t2, start):
            ev = e_v[pl.ds((first_own + t2) * _L, _L)]
            pos_vec = jnp.zeros((_L,), jnp.int32)
            for v in range(E):
                m = ev == v
                mi = jnp.where(m, jnp.int32(1), jnp.int32(0))
                incl = plsc.cumsum(mi)
                base_v = jnp.sum(start * onehots[v])
                pos_vec = jnp.where(m, base_v + incl - 1, pos_vec)
                cnt = plsc.all_reduce_population_count(m)
                start = start + cnt * onehots[v]
            pos_v[pl.ds(t2 * _L, _L)] = pos_vec
            return start

        lax.fori_loop(0, n_vec_chunk, body, start0)

        pltpu.sync_copy(pos_v, pos_hbm.at[pl.ds(w * chunk, chunk)])

    return sort_kernel


def _make_gather_kernel(M, H, start, Mc):
    rows_w = Mc // _NW
    cb = 16
    n_cb = rows_w // cb
    n_vec_total = M // _L
    mesh = plsc.VectorSubcoreMesh(core_axis_name="c", subcore_axis_name="s")

    @functools.partial(
        pl.kernel,
        out_type=jax.ShapeDtypeStruct((Mc, H), jnp.float32),
        mesh=mesh,
        scratch_types=(
            [pltpu.VMEM((M,), jnp.int32),
             pltpu.VMEM((rows_w,), jnp.int32)]
            + [pltpu.VMEM((cb, H), jnp.float32)] * 2
            + [pltpu.VMEM((cb,), jnp.int32)] * 2
            + [pltpu.SemaphoreType.DMA] * 4
        ),
        compiler_params=_sc_compiler_params(),
    )
    def gather_kernel(x_hbm, pos_hbm, xs_hbm, pos_v, perm_v, *rest):
        bufs = rest[0:2]
        idxs = rest[2:4]
        gsems = rest[4:6]
        ssems = rest[6:8]
        w = _wid()
        lanes = lax.iota(jnp.int32, _L)
        base = start + w * rows_w
        pltpu.sync_copy(pos_hbm, pos_v)

        @pl.loop(0, n_vec_total)
        def _(t):
            pv = pos_v[pl.ds(t * _L, _L)]
            rel = pv - base
            m = (rel >= 0) & (rel < rows_w)
            relc = jnp.where(m, rel, 0)
            plsc.store_scatter(perm_v, [relc], lanes + t * _L, mask=m)

        gathers = [None, None]
        stores = [None] * n_cb
        for c in range(min(2, n_cb)):
            idxs[c][...] = perm_v[pl.ds(c * cb, cb)]
            gathers[c] = pltpu.async_copy(
                x_hbm.at[idxs[c]], bufs[c], gsems[c])
        for c in range(n_cb):
            b = c & 1
            gathers[b].wait()
            stores[c] = pltpu.async_copy(
                bufs[b], xs_hbm.at[pl.ds(w * rows_w + c * cb, cb)], ssems[b])
            nxt = c + 2
            if nxt < n_cb:
                stores[c].wait()
                idxs[b][...] = perm_v[pl.ds(nxt * cb, cb)]
                gathers[b] = pltpu.async_copy(
                    x_hbm.at[idxs[b]], bufs[b], gsems[b])
        for c in range(max(0, n_cb - 2), n_cb):
            if stores[c] is not None:
                stores[c].wait()

    return gather_kernel


def _mm_body(x_ref, w_ref, be_ref, s_ref, o_ref):
    xb = x_ref[...].astype(jnp.bfloat16)
    acc = jnp.dot(xb, w_ref[...], preferred_element_type=jnp.float32)
    o_ref[...] = (acc + be_ref[...]) * s_ref[...]


def _mm_body_acc(x_ref, w_ref, be_ref, s_ref, prev_ref, o_ref):
    del prev_ref
    _mm_body(x_ref, w_ref, be_ref, s_ref, o_ref)


def _expert_matmul_chunk(xs, We_bf, be2, p2, prev, start, bm=1024):
    Mc, H = xs.shape
    M = p2.shape[0]
    blocks = Mc // bm
    b0 = start // bm
    in_specs = [
        pl.BlockSpec((bm, H), lambda i: (i, 0)),
        pl.BlockSpec((H, H), lambda i: (0, 0)),
        pl.BlockSpec((1, H), lambda i: (0, 0)),
        pl.BlockSpec((bm, 1), lambda i, b0=b0: (b0 + i, 0)),
    ]
    args = [xs, We_bf, be2, p2]
    if prev is None:
        body = _mm_body
        aliases = {}
    else:
        body = _mm_body_acc
        in_specs.append(pl.BlockSpec(memory_space=pl.ANY))
        args.append(prev)
        aliases = {4: 0}
    return pl.pallas_call(
        body,
        grid=(blocks,),
        in_specs=in_specs,
        out_specs=pl.BlockSpec((bm, H), lambda i, b0=b0: (b0 + i, 0)),
        out_shape=jax.ShapeDtypeStruct((M, H), jnp.float32),
        input_output_aliases=aliases,
    )(*args)


def kernel(x, Wg, bg, We, be):
    B, S, H = x.shape
    E = Wg.shape[1]
    M = B * S
    hs = x.reshape(M, H)

    router_logits = hs @ Wg + bg
    normalized_logits = jax.nn.softmax(router_logits, axis=1)
    best = jnp.argmax(normalized_logits, axis=1)
    p = jnp.max(normalized_logits, axis=1)

    e = best.astype(jnp.int32)
    pos = _make_sort_kernel(M, E)(e)
    We_bf = We.astype(jnp.bfloat16)
    be2 = be.reshape(1, H)
    p2 = p.reshape(M, 1)

    out = None
    start = 0
    for Mc in _CHUNKS:
        xs_k = _make_gather_kernel(M, H, start, Mc)(hs, pos)
        out = _expert_matmul_chunk(xs_k, We_bf, be2, p2, out, start)
        start += Mc
    return out.reshape(B, S, H)
